# Initial kernel scaffold; baseline (speedup 1.0000x reference)
import functools

import jax
import jax.numpy as jnp
from jax import lax
from jax.experimental import pallas as pl
from jax.experimental.pallas import tpu as pltpu
from jax.experimental.pallas import tpu_sc as plsc

_N = 10000
_E = 320000
_D = 128
_G = 64
_NP = 10240
_NC = 2
_NS = 16
_NW = _NC * _NS
_EW = _E // _NW
_C = 125
_NCH = _EW // _C
_RPT = _NP // _NS
_BLK = 512
_NBLK = _NP // _BLK


def _mm_body(x_ref, w_ref, o_ref):
    o_ref[0] = jnp.dot(x_ref[...], w_ref[0], preferred_element_type=jnp.float32)


def _rel_transform(x, w_ext):
    return pl.pallas_call(
        _mm_body,
        grid=(_NBLK, 4),
        in_specs=[
            pl.BlockSpec((_BLK, _D), lambda i, r: (i, 0)),
            pl.BlockSpec((1, _D, _D), lambda i, r: (r, 0, 0)),
        ],
        out_specs=pl.BlockSpec((1, _BLK, _D), lambda i, r: (r, i, 0)),
        out_shape=jax.ShapeDtypeStruct((4, _NP, _D), jnp.float32),
    )(x, w_ext)


def _sc_body(table, gidx_hbm, didx_hbm, didxf_hbm, acc_out, cnt_out,
             gidx_v, didx_v, didx_flat, rowbuf, zbuf, cntp, acc_sh, sem0):
    c = lax.axis_index("c")
    s = lax.axis_index("s")
    wid = s * _NC + c

    pltpu.sync_copy(gidx_hbm.at[wid], gidx_v)
    pltpu.sync_copy(didx_hbm.at[wid], didx_v)
    pltpu.sync_copy(didxf_hbm.at[wid], didx_flat)

    zero16 = jnp.zeros((16,), jnp.float32)

    def _zrow(r, carry):
        def _zcol(k, carry2):
            zbuf[r, pl.ds(k * 16, 16)] = zero16
            return carry2
        return lax.fori_loop(0, _D // 16, _zcol, carry)
    lax.fori_loop(0, 128, _zrow, 0)

    def _zcnt(t, carry):
        cntp[pl.ds(t * 16, 16)] = zero16
        return carry
    lax.fori_loop(0, _NP // 16, _zcnt, 0)

    rows0 = s * _RPT
    for q in range(_RPT // 128):
        pltpu.sync_copy(zbuf, acc_sh.at[pl.ds(rows0 + q * 128, 128)])
    plsc.subcore_barrier()

    def _chunk(j, carry):
        pltpu.async_copy(table.at[gidx_v.at[j]], rowbuf, sem0).wait()
        pltpu.sync_copy(rowbuf, acc_sh.at[didx_v.at[j]], add=True)
        return carry
    lax.fori_loop(0, _NCH, _chunk, 0)

    ones16 = jnp.ones((16,), jnp.float32)

    def _cnt(t, carry):
        idx = didx_flat[pl.ds(t * 16, 16)]
        plsc.addupdate_scatter(cntp, [idx], ones16)
        return carry
    lax.fori_loop(0, _EW // 16, _cnt, 0)
    pltpu.sync_copy(cntp, cnt_out.at[wid])

    plsc.subcore_barrier()

    for q in range(_RPT // 128):
        pltpu.sync_copy(acc_sh.at[pl.ds(rows0 + q * 128, 128)], zbuf)
        pltpu.sync_copy(zbuf, acc_out.at[pl.ds(c * _NP + rows0 + q * 128, 128)])


def _sc_aggregate(table_flat, gidx, didx, didxf):
    mesh = plsc.VectorSubcoreMesh(core_axis_name="c", subcore_axis_name="s")
    k = functools.partial(
        pl.kernel,
        mesh=mesh,
        out_type=[
            jax.ShapeDtypeStruct((_NC * _NP, _D), jnp.float32),
            jax.ShapeDtypeStruct((_NW, _NP), jnp.float32),
        ],
        scratch_types=[
            pltpu.VMEM((_NCH, _C), jnp.int32),
            pltpu.VMEM((_NCH, _C), jnp.int32),
            pltpu.VMEM((_EW,), jnp.int32),
            pltpu.VMEM((_C, _D), jnp.float32),
            pltpu.VMEM((128, _D), jnp.float32),
            pltpu.VMEM((_NP,), jnp.float32),
            pltpu.VMEM_SHARED((_NP, _D), jnp.float32),
            pltpu.SemaphoreType.DMA,
        ],
    )(_sc_body)
    return k(table_flat, gidx, didx, didxf)


def _norm_body(acc_ref, cntp_ref, xr_ref, b_ref, h_ref, cnt_ref):
    cnt = jnp.sum(cntp_ref[...], axis=0)
    mean = (acc_ref[0] + acc_ref[1]) / jnp.maximum(cnt, 1.0)[:, None]
    h_ref[...] = jnp.maximum(mean + xr_ref[0] + b_ref[...], 0.0)
    cnt_ref[...] = cnt[None, :]


def _normalize(acc, cntp, t, b):
    return pl.pallas_call(
        _norm_body,
        grid=(_NBLK,),
        in_specs=[
            pl.BlockSpec((2, _BLK, _D), lambda i: (0, i, 0)),
            pl.BlockSpec((_NW, _BLK), lambda i: (0, i)),
            pl.BlockSpec((1, _BLK, _D), lambda i: (3, i, 0)),
            pl.BlockSpec((1, _D), lambda i: (0, 0)),
        ],
        out_specs=[
            pl.BlockSpec((_BLK, _D), lambda i: (i, 0)),
            pl.BlockSpec((1, _BLK), lambda i: (0, i)),
        ],
        out_shape=[
            jax.ShapeDtypeStruct((_NP, _D), jnp.float32),
            jax.ShapeDtypeStruct((1, _NP), jnp.float32),
        ],
    )(acc.reshape(2, _NP, _D), cntp, t, b)


def _pool_body(acc_ref, cnt_ref, hr_ref, b_ref, batch_ref, out_ref, counts_ref):
    i = pl.program_id(0)
    cnt = cnt_ref[0]
    h2 = jnp.maximum(
        (acc_ref[0] + acc_ref[1]) / jnp.maximum(cnt, 1.0)[:, None]
        + hr_ref[0] + b_ref[...], 0.0)
    gid = lax.broadcasted_iota(jnp.int32, (_G, _BLK), 0)
    onehot = (batch_ref[0, 0][None, :] == gid).astype(jnp.float32)

    @pl.when(i == 0)
    def _():
        out_ref[...] = jnp.zeros_like(out_ref)
        counts_ref[...] = jnp.zeros_like(counts_ref)

    out_ref[...] += jnp.dot(onehot, h2, preferred_element_type=jnp.float32)
    counts_ref[...] += jnp.sum(onehot, axis=1)[:, None]

    @pl.when(i == pl.num_programs(0) - 1)
    def _():
        out_ref[...] = out_ref[...] / jnp.maximum(counts_ref[...], 1.0)


def _pool(acc, cnt2d, t, b, batch3d):
    return pl.pallas_call(
        _pool_body,
        grid=(_NBLK,),
        in_specs=[
            pl.BlockSpec((2, _BLK, _D), lambda i: (0, i, 0)),
            pl.BlockSpec((1, _BLK), lambda i: (0, i)),
            pl.BlockSpec((1, _BLK, _D), lambda i: (3, i, 0)),
            pl.BlockSpec((1, _D), lambda i: (0, 0)),
            pl.BlockSpec((1, 1, _BLK), lambda i: (i, 0, 0)),
        ],
        out_specs=pl.BlockSpec((_G, _D), lambda i: (0, 0)),
        out_shape=jax.ShapeDtypeStruct((_G, _D), jnp.float32),
        scratch_shapes=[pltpu.VMEM((_G, _D), jnp.float32)],
    )(acc.reshape(2, _NP, _D), cnt2d, t, b, batch3d)


def kernel(x, edge_index, edge_type, batch, W1, root1, b1, W2, root2, b2):
    xp = jnp.pad(x, ((0, _NP - _N), (0, 0)))
    src = edge_index[0].astype(jnp.int32)
    dst = edge_index[1].astype(jnp.int32)
    et = edge_type.astype(jnp.int32)
    gidx = (et * _NP + src).reshape(_NW, _NCH, _C)
    didx = dst.reshape(_NW, _NCH, _C)
    didxf = dst.reshape(_NW, _EW)
    batch3d = jnp.concatenate(
        [batch.astype(jnp.int32), jnp.full((_NP - _N,), _G, jnp.int32)]
    ).reshape(_NBLK, 1, _BLK)
    w1e = jnp.concatenate([W1, root1[None]], axis=0)
    w2e = jnp.concatenate([W2, root2[None]], axis=0)
    b1r = b1.reshape(1, _D)
    b2r = b2.reshape(1, _D)

    t1 = _rel_transform(xp, w1e)
    acc1, cntp = _sc_aggregate(t1.reshape(4 * _NP, _D), gidx, didx, didxf)
    h, cnt2d = _normalize(acc1, cntp, t1, b1r)
    t2 = _rel_transform(h, w2e)
    acc2, _ = _sc_aggregate(t2.reshape(4 * _NP, _D), gidx, didx, didxf)
    return _pool(acc2, cnt2d, t2, b2r, batch3d)


# speedup vs baseline: 8.9543x; 8.9543x over previous
import functools

import jax
import jax.numpy as jnp
from jax import lax
from jax.experimental import pallas as pl
from jax.experimental.pallas import tpu as pltpu
from jax.experimental.pallas import tpu_sc as plsc

_N = 10000
_E = 320000
_D = 128
_G = 64
_NP = 10240
_NC = 2
_NS = 16
_NW = _NC * _NS
_EW = _E // _NW
_C = 128
_NCH = 80
_EP = _NW * _NCH * _C
_RPT = _NP // _NS
_BLK = 512
_NBLK = _NP // _BLK


def _mm_body(x_ref, w_ref, o_ref):
    o_ref[0] = jnp.dot(x_ref[...], w_ref[0], preferred_element_type=jnp.float32)


def _rel_transform(x, w_ext):
    return pl.pallas_call(
        _mm_body,
        grid=(_NBLK, 4),
        in_specs=[
            pl.BlockSpec((_BLK, _D), lambda i, r: (i, 0)),
            pl.BlockSpec((1, _D, _D), lambda i, r: (r, 0, 0)),
        ],
        out_specs=pl.BlockSpec((1, _BLK, _D), lambda i, r: (r, i, 0)),
        out_shape=jax.ShapeDtypeStruct((4, _NP, _D), jnp.float32),
    )(x, w_ext)


_QCH = _NCH // 5


def _sc_body(table, pidx_hbm, acc_out,
             pbuf, gbuf, dbuf, rowbuf, acc_sh, sem0):
    c = lax.axis_index("c")
    s = lax.axis_index("s")
    wid = s * _NC + c

    zero16 = jnp.zeros((16,), jnp.float32)

    def _zrow(r, carry):
        def _zcol(k, carry2):
            rowbuf[r, pl.ds(k * 16, 16)] = zero16
            return carry2
        return lax.fori_loop(0, _D // 16, _zcol, carry)
    lax.fori_loop(0, _C, _zrow, 0)

    rows0 = s * _RPT
    for q in range(_RPT // _C):
        pltpu.sync_copy(rowbuf, acc_sh.at[pl.ds(rows0 + q * _C, _C)])
    plsc.subcore_barrier()

    def _chunk(j, carry):
        pltpu.async_copy(table.at[gbuf.at[j]], rowbuf, sem0).wait()
        pltpu.sync_copy(rowbuf, acc_sh.at[dbuf.at[j]], add=True)
        return carry

    def _unpack(t, carry):
        r = t // (_C // 16)
        k = t % (_C // 16)
        v = pbuf[r, pl.ds(k * 16, 16)]
        gbuf[r, pl.ds(k * 16, 16)] = lax.shift_right_logical(v, 14)
        dbuf[r, pl.ds(k * 16, 16)] = lax.bitwise_and(v, 16383)
        return carry

    for slab in range(5):
        pltpu.sync_copy(pidx_hbm.at[wid, pl.ds(slab * _QCH, _QCH)], pbuf)
        lax.fori_loop(0, _QCH * (_C // 16), _unpack, 0)
        lax.fori_loop(0, _QCH, _chunk, 0)

    plsc.subcore_barrier()

    for q in range(_RPT // _C):
        pltpu.sync_copy(acc_sh.at[pl.ds(rows0 + q * _C, _C)], rowbuf)
        pltpu.sync_copy(rowbuf, acc_out.at[pl.ds(c * _NP + rows0 + q * _C, _C)])


def _sc_aggregate(table_flat, pidx):
    mesh = plsc.VectorSubcoreMesh(core_axis_name="c", subcore_axis_name="s")
    k = functools.partial(
        pl.kernel,
        mesh=mesh,
        out_type=jax.ShapeDtypeStruct((_NC * _NP, _D), jnp.float32),
        scratch_types=[
            pltpu.VMEM((_QCH, _C), jnp.int32),
            pltpu.VMEM((_QCH, _C), jnp.int32),
            pltpu.VMEM((_QCH, _C), jnp.int32),
            pltpu.VMEM((_C, _D), jnp.float32),
            pltpu.VMEM_SHARED((_NP, _D), jnp.float32),
            pltpu.SemaphoreType.DMA,
        ],
    )(_sc_body)
    return k(table_flat, pidx)


def _cnt_body(pidx_hbm, cnt_out, pbuf, dbuf, onesbuf, cbuf, ccomp, cnt_sh):
    c = lax.axis_index("c")
    s = lax.axis_index("s")
    wid = s * _NC + c

    zero16 = jnp.zeros((16,), jnp.float32)
    ones16 = jnp.ones((16,), jnp.float32)

    def _zc(r, carry):
        def _zk(k, carry2):
            cbuf[r, pl.ds(k * 16, 16)] = zero16
            onesbuf[r, pl.ds(k * 16, 16)] = ones16
            return carry2
        return lax.fori_loop(0, _D // 16, _zk, carry)
    lax.fori_loop(0, _C, _zc, 0)

    rows0 = s * _RPT
    for q in range(_RPT // _C):
        pltpu.sync_copy(cbuf, cnt_sh.at[pl.ds(rows0 + q * _C, _C)])
    plsc.subcore_barrier()

    def _chunk(j, carry):
        pltpu.sync_copy(onesbuf, cnt_sh.at[dbuf.at[j]], add=True)
        return carry

    def _unpack(t, carry):
        r = t // (_C // 16)
        k = t % (_C // 16)
        v = pbuf[r, pl.ds(k * 16, 16)]
        dbuf[r, pl.ds(k * 16, 16)] = lax.bitwise_and(v, 16383)
        return carry

    for slab in range(5):
        pltpu.sync_copy(pidx_hbm.at[wid, pl.ds(slab * _QCH, _QCH)], pbuf)
        lax.fori_loop(0, _QCH * (_C // 16), _unpack, 0)
        lax.fori_loop(0, _QCH, _chunk, 0)

    plsc.subcore_barrier()

    for q in range(_RPT // _C):
        pltpu.sync_copy(cnt_sh.at[pl.ds(rows0 + q * _C, _C)], cbuf)

        def _ext(g, carry):
            lane = lax.iota(jnp.int32, 16)
            w = jnp.zeros((16,), jnp.float32)
            for i in range(16):
                v = cbuf[g * 16 + i, pl.ds(0, 16)]
                w = jnp.where(lane == i, v, w)
            ccomp[pl.ds(q * _C + g * 16, 16)] = w
            return carry
        lax.fori_loop(0, _C // 16, _ext, 0)
    pltpu.sync_copy(ccomp, cnt_out.at[pl.ds(c * _NP + rows0, _RPT)])


def _sc_count(pidx):
    mesh = plsc.VectorSubcoreMesh(core_axis_name="c", subcore_axis_name="s")
    k = functools.partial(
        pl.kernel,
        mesh=mesh,
        out_type=jax.ShapeDtypeStruct((_NC * _NP,), jnp.float32),
        scratch_types=[
            pltpu.VMEM((_QCH, _C), jnp.int32),
            pltpu.VMEM((_QCH, _C), jnp.int32),
            pltpu.VMEM((_C, _D), jnp.float32),
            pltpu.VMEM((_C, _D), jnp.float32),
            pltpu.VMEM((_RPT,), jnp.float32),
            pltpu.VMEM_SHARED((_NP, _D), jnp.float32),
        ],
    )(_cnt_body)
    return k(pidx)


def _norm_body(acc_ref, cntp_ref, xr_ref, b_ref, h_ref, cnt_ref):
    i = pl.program_id(0)
    row = lax.broadcasted_iota(jnp.int32, (_BLK, 1), 0) + i * _BLK
    cnt = cntp_ref[0] + cntp_ref[1]
    mean = (acc_ref[0] + acc_ref[1]) / jnp.maximum(cnt, 1.0)[:, None]
    h = jnp.maximum(mean + xr_ref[0] + b_ref[...], 0.0)
    h_ref[...] = jnp.where(row < _N, h, 0.0)
    cnt_ref[...] = cnt[None, :]


def _normalize(acc, cntp, t, b):
    return pl.pallas_call(
        _norm_body,
        grid=(_NBLK,),
        in_specs=[
            pl.BlockSpec((2, _BLK, _D), lambda i: (0, i, 0)),
            pl.BlockSpec((2, _BLK), lambda i: (0, i)),
            pl.BlockSpec((1, _BLK, _D), lambda i: (3, i, 0)),
            pl.BlockSpec((1, _D), lambda i: (0, 0)),
        ],
        out_specs=[
            pl.BlockSpec((_BLK, _D), lambda i: (i, 0)),
            pl.BlockSpec((1, _BLK), lambda i: (0, i)),
        ],
        out_shape=[
            jax.ShapeDtypeStruct((_NP, _D), jnp.float32),
            jax.ShapeDtypeStruct((1, _NP), jnp.float32),
        ],
    )(acc.reshape(2, _NP, _D), cntp.reshape(2, _NP), t, b)


def _pool_body(acc_ref, cnt_ref, hr_ref, b_ref, batch_ref, out_ref, counts_ref):
    i = pl.program_id(0)
    row = lax.broadcasted_iota(jnp.int32, (_BLK, 1), 0) + i * _BLK
    cnt = cnt_ref[0]
    h2 = jnp.maximum(
        (acc_ref[0] + acc_ref[1]) / jnp.maximum(cnt, 1.0)[:, None]
        + hr_ref[0] + b_ref[...], 0.0)
    h2 = jnp.where(row < _N, h2, 0.0)
    gid = lax.broadcasted_iota(jnp.int32, (_G, _BLK), 0)
    onehot = (batch_ref[0, 0][None, :] == gid).astype(jnp.float32)

    @pl.when(i == 0)
    def _():
        out_ref[...] = jnp.zeros_like(out_ref)
        counts_ref[...] = jnp.zeros_like(counts_ref)

    out_ref[...] += jnp.dot(onehot, h2, preferred_element_type=jnp.float32)
    counts_ref[...] += jnp.sum(onehot, axis=1)[:, None]

    @pl.when(i == pl.num_programs(0) - 1)
    def _():
        out_ref[...] = out_ref[...] / jnp.maximum(counts_ref[...], 1.0)


def _pool(acc, cnt2d, t, b, batch3d):
    return pl.pallas_call(
        _pool_body,
        grid=(_NBLK,),
        in_specs=[
            pl.BlockSpec((2, _BLK, _D), lambda i: (0, i, 0)),
            pl.BlockSpec((1, _BLK), lambda i: (0, i)),
            pl.BlockSpec((1, _BLK, _D), lambda i: (3, i, 0)),
            pl.BlockSpec((1, _D), lambda i: (0, 0)),
            pl.BlockSpec((1, 1, _BLK), lambda i: (i, 0, 0)),
        ],
        out_specs=pl.BlockSpec((_G, _D), lambda i: (0, 0)),
        out_shape=jax.ShapeDtypeStruct((_G, _D), jnp.float32),
        scratch_shapes=[pltpu.VMEM((_G, _D), jnp.float32)],
    )(acc.reshape(2, _NP, _D), cnt2d, t, b, batch3d)


def kernel(x, edge_index, edge_type, batch, W1, root1, b1, W2, root2, b2):
    xp = jnp.pad(x, ((0, _NP - _N), (0, 0)))
    src = edge_index[0].astype(jnp.int32)
    dst = edge_index[1].astype(jnp.int32)
    et = edge_type.astype(jnp.int32)
    packed = (et * _NP + src) * 16384 + dst
    pidx = jnp.concatenate(
        [packed, jnp.full((_EP - _E,), _NP - 1, jnp.int32)]
    ).reshape(_NW, _NCH, _C)
    batch3d = jnp.concatenate(
        [batch.astype(jnp.int32), jnp.full((_NP - _N,), _G, jnp.int32)]
    ).reshape(_NBLK, 1, _BLK)
    w1e = jnp.concatenate([W1, root1[None]], axis=0)
    w2e = jnp.concatenate([W2, root2[None]], axis=0)
    b1r = b1.reshape(1, _D)
    b2r = b2.reshape(1, _D)

    cntp = _sc_count(pidx)
    t1 = _rel_transform(xp, w1e)
    acc1 = _sc_aggregate(t1.reshape(4 * _NP, _D), pidx)
    h, cnt2d = _normalize(acc1, cntp, t1, b1r)
    t2 = _rel_transform(h, w2e)
    acc2 = _sc_aggregate(t2.reshape(4 * _NP, _D), pidx)
    return _pool(acc2, cnt2d, t2, b2r, batch3d)


# speedup vs baseline: 10.5624x; 1.1796x over previous
import functools

import jax
import jax.numpy as jnp
from jax import lax
from jax.experimental import pallas as pl
from jax.experimental.pallas import tpu as pltpu
from jax.experimental.pallas import tpu_sc as plsc

_N = 10000
_E = 320000
_D = 128
_G = 64
_NP = 10240
_NC = 2
_NS = 16
_NW = _NC * _NS
_EW = _E // _NW
_C = 64
_NCH = 160
_EP = _NW * _NCH * _C
_RPT = _NP // _NS
_BLK = 512
_NBLK = _NP // _BLK


def _mm_body(x_ref, w_ref, o_ref):
    o_ref[0] = jnp.dot(x_ref[...], w_ref[0], preferred_element_type=jnp.float32)


def _rel_transform(x, w_ext):
    return pl.pallas_call(
        _mm_body,
        grid=(_NBLK, 4),
        in_specs=[
            pl.BlockSpec((_BLK, _D), lambda i, r: (i, 0)),
            pl.BlockSpec((1, _D, _D), lambda i, r: (r, 0, 0)),
        ],
        out_specs=pl.BlockSpec((1, _BLK, _D), lambda i, r: (r, i, 0)),
        out_shape=jax.ShapeDtypeStruct((4, _NP, _D), jnp.float32),
    )(x, w_ext)


_QCH = _NCH // 5


_HCH = _NCH // 2


def _sc_body(table, pidx_hbm, acc_out,
             pkbuf, gb2, db2, rowbuf, acc_sh, sem0, sem1):
    c = lax.axis_index("c")
    s = lax.axis_index("s")
    wid = s * _NC + c

    zero16 = jnp.zeros((16,), jnp.float32)

    def _zrow(r, carry):
        def _zcol(k, carry2):
            rowbuf[0, r, pl.ds(k * 16, 16)] = zero16
            return carry2
        return lax.fori_loop(0, _D // 16, _zcol, carry)
    lax.fori_loop(0, _C, _zrow, 0)

    rows0 = s * _RPT
    for q in range(_RPT // _C):
        pltpu.sync_copy(rowbuf.at[0], acc_sh.at[pl.ds(rows0 + q * _C, _C)])
    plsc.subcore_barrier()

    def _unpack_to(r, b):
        for k in range(_C // 16):
            v = pkbuf[r, pl.ds(k * 16, 16)]
            gb2[b, pl.ds(k * 16, 16)] = lax.shift_right_logical(v, 14)
            db2[b, pl.ds(k * 16, 16)] = lax.bitwise_and(v, 16383)

    def _gather(b, sem):
        pltpu.async_copy(table.at[gb2.at[b]], rowbuf.at[b], sem)

    def _gwait(b, sem):
        pltpu.make_async_copy(table.at[gb2.at[b]], rowbuf.at[b], sem).wait()

    def _scatter(b):
        pltpu.sync_copy(rowbuf.at[b], acc_sh.at[db2.at[b]], add=True)

    for half in range(2):
        pltpu.sync_copy(pidx_hbm.at[wid, pl.ds(half * _HCH, _HCH)], pkbuf)
        _unpack_to(0, 0)
        _gather(0, sem0)

        def _pair(t, carry):
            r0 = 2 * t
            _unpack_to(r0 + 1, 1)
            _gather(1, sem1)
            _gwait(0, sem0)
            _scatter(0)

            @pl.when(r0 + 2 < _HCH)
            def _():
                _unpack_to(r0 + 2, 0)
                _gather(0, sem0)
            _gwait(1, sem1)
            _scatter(1)
            return carry
        lax.fori_loop(0, _HCH // 2, _pair, 0)

    plsc.subcore_barrier()

    for q in range(_RPT // _C):
        pltpu.sync_copy(acc_sh.at[pl.ds(rows0 + q * _C, _C)], rowbuf.at[0])
        pltpu.sync_copy(rowbuf.at[0],
                        acc_out.at[pl.ds(c * _NP + rows0 + q * _C, _C)])


def _sc_aggregate(table_flat, pidx):
    mesh = plsc.VectorSubcoreMesh(core_axis_name="c", subcore_axis_name="s")
    k = functools.partial(
        pl.kernel,
        mesh=mesh,
        out_type=jax.ShapeDtypeStruct((_NC * _NP, _D), jnp.float32),
        scratch_types=[
            pltpu.VMEM((_HCH, _C), jnp.int32),
            pltpu.VMEM((2, _C), jnp.int32),
            pltpu.VMEM((2, _C), jnp.int32),
            pltpu.VMEM((2, _C, _D), jnp.float32),
            pltpu.VMEM_SHARED((_NP, _D), jnp.float32),
            pltpu.SemaphoreType.DMA,
            pltpu.SemaphoreType.DMA,
        ],
    )(_sc_body)
    return k(table_flat, pidx)


def _cnt_body(pidx_hbm, cnt_out, dbuf, onesbuf, cbuf, ccomp, cnt_sh):
    c = lax.axis_index("c")
    s = lax.axis_index("s")
    wid = s * _NC + c

    zero16 = jnp.zeros((16,), jnp.float32)
    ones16 = jnp.ones((16,), jnp.float32)

    def _zc(r, carry):
        def _zk(k, carry2):
            cbuf[r, pl.ds(k * 16, 16)] = zero16
            onesbuf[r, pl.ds(k * 16, 16)] = ones16
            return carry2
        return lax.fori_loop(0, _D // 16, _zk, carry)
    lax.fori_loop(0, _C, _zc, 0)

    rows0 = s * _RPT
    for q in range(_RPT // _C):
        pltpu.sync_copy(cbuf, cnt_sh.at[pl.ds(rows0 + q * _C, _C)])
    plsc.subcore_barrier()

    def _chunk(j, carry):
        pltpu.sync_copy(onesbuf, cnt_sh.at[dbuf.at[j]], add=True)
        return carry

    def _unpack(t, carry):
        r = t // (_C // 16)
        k = t % (_C // 16)
        v = dbuf[r, pl.ds(k * 16, 16)]
        dbuf[r, pl.ds(k * 16, 16)] = lax.bitwise_and(v, 16383)
        return carry

    pltpu.sync_copy(pidx_hbm.at[wid], dbuf)
    lax.fori_loop(0, _NCH * (_C // 16), _unpack, 0)
    lax.fori_loop(0, _NCH, _chunk, 0)

    plsc.subcore_barrier()

    for q in range(_RPT // _C):
        pltpu.sync_copy(cnt_sh.at[pl.ds(rows0 + q * _C, _C)], cbuf)

        def _ext(g, carry):
            lane = lax.iota(jnp.int32, 16)
            w = jnp.zeros((16,), jnp.float32)
            for i in range(16):
                v = cbuf[g * 16 + i, pl.ds(0, 16)]
                w = jnp.where(lane == i, v, w)
            ccomp[pl.ds(q * _C + g * 16, 16)] = w
            return carry
        lax.fori_loop(0, _C // 16, _ext, 0)
    pltpu.sync_copy(ccomp, cnt_out.at[pl.ds(c * _NP + rows0, _RPT)])


def _sc_count(pidx):
    mesh = plsc.VectorSubcoreMesh(core_axis_name="c", subcore_axis_name="s")
    k = functools.partial(
        pl.kernel,
        mesh=mesh,
        out_type=jax.ShapeDtypeStruct((_NC * _NP,), jnp.float32),
        scratch_types=[
            pltpu.VMEM((_NCH, _C), jnp.int32),
            pltpu.VMEM((_C, _D), jnp.float32),
            pltpu.VMEM((_C, _D), jnp.float32),
            pltpu.VMEM((_RPT,), jnp.float32),
            pltpu.VMEM_SHARED((_NP, _D), jnp.float32),
        ],
    )(_cnt_body)
    return k(pidx)


def _norm_body(acc_ref, cntp_ref, xr_ref, b_ref, h_ref, cnt_ref):
    i = pl.program_id(0)
    row = lax.broadcasted_iota(jnp.int32, (_BLK, 1), 0) + i * _BLK
    cnt = cntp_ref[0] + cntp_ref[1]
    mean = (acc_ref[0] + acc_ref[1]) / jnp.maximum(cnt, 1.0)[:, None]
    h = jnp.maximum(mean + xr_ref[0] + b_ref[...], 0.0)
    h_ref[...] = jnp.where(row < _N, h, 0.0)
    cnt_ref[...] = cnt[None, :]


def _normalize(acc, cntp, t, b):
    return pl.pallas_call(
        _norm_body,
        grid=(_NBLK,),
        in_specs=[
            pl.BlockSpec((2, _BLK, _D), lambda i: (0, i, 0)),
            pl.BlockSpec((2, _BLK), lambda i: (0, i)),
            pl.BlockSpec((1, _BLK, _D), lambda i: (3, i, 0)),
            pl.BlockSpec((1, _D), lambda i: (0, 0)),
        ],
        out_specs=[
            pl.BlockSpec((_BLK, _D), lambda i: (i, 0)),
            pl.BlockSpec((1, _BLK), lambda i: (0, i)),
        ],
        out_shape=[
            jax.ShapeDtypeStruct((_NP, _D), jnp.float32),
            jax.ShapeDtypeStruct((1, _NP), jnp.float32),
        ],
    )(acc.reshape(2, _NP, _D), cntp.reshape(2, _NP), t, b)


def _pool_body(acc_ref, cnt_ref, hr_ref, b_ref, batch_ref, out_ref, counts_ref):
    i = pl.program_id(0)
    row = lax.broadcasted_iota(jnp.int32, (_BLK, 1), 0) + i * _BLK
    cnt = cnt_ref[0]
    h2 = jnp.maximum(
        (acc_ref[0] + acc_ref[1]) / jnp.maximum(cnt, 1.0)[:, None]
        + hr_ref[0] + b_ref[...], 0.0)
    h2 = jnp.where(row < _N, h2, 0.0)
    gid = lax.broadcasted_iota(jnp.int32, (_G, _BLK), 0)
    onehot = (batch_ref[0, 0][None, :] == gid).astype(jnp.float32)

    @pl.when(i == 0)
    def _():
        out_ref[...] = jnp.zeros_like(out_ref)
        counts_ref[...] = jnp.zeros_like(counts_ref)

    out_ref[...] += jnp.dot(onehot, h2, preferred_element_type=jnp.float32)
    counts_ref[...] += jnp.sum(onehot, axis=1)[:, None]

    @pl.when(i == pl.num_programs(0) - 1)
    def _():
        out_ref[...] = out_ref[...] / jnp.maximum(counts_ref[...], 1.0)


def _pool(acc, cnt2d, t, b, batch3d):
    return pl.pallas_call(
        _pool_body,
        grid=(_NBLK,),
        in_specs=[
            pl.BlockSpec((2, _BLK, _D), lambda i: (0, i, 0)),
            pl.BlockSpec((1, _BLK), lambda i: (0, i)),
            pl.BlockSpec((1, _BLK, _D), lambda i: (3, i, 0)),
            pl.BlockSpec((1, _D), lambda i: (0, 0)),
            pl.BlockSpec((1, 1, _BLK), lambda i: (i, 0, 0)),
        ],
        out_specs=pl.BlockSpec((_G, _D), lambda i: (0, 0)),
        out_shape=jax.ShapeDtypeStruct((_G, _D), jnp.float32),
        scratch_shapes=[pltpu.VMEM((_G, _D), jnp.float32)],
    )(acc.reshape(2, _NP, _D), cnt2d, t, b, batch3d)


def kernel(x, edge_index, edge_type, batch, W1, root1, b1, W2, root2, b2):
    xp = jnp.pad(x, ((0, _NP - _N), (0, 0)))
    src = edge_index[0].astype(jnp.int32)
    dst = edge_index[1].astype(jnp.int32)
    et = edge_type.astype(jnp.int32)
    packed = (et * _NP + src) * 16384 + dst
    pidx = jnp.concatenate(
        [packed, jnp.full((_EP - _E,), _NP - 1, jnp.int32)]
    ).reshape(_NW, _NCH, _C)
    batch3d = jnp.concatenate(
        [batch.astype(jnp.int32), jnp.full((_NP - _N,), _G, jnp.int32)]
    ).reshape(_NBLK, 1, _BLK)
    w1e = jnp.concatenate([W1, root1[None]], axis=0)
    w2e = jnp.concatenate([W2, root2[None]], axis=0)
    b1r = b1.reshape(1, _D)
    b2r = b2.reshape(1, _D)

    cntp = _sc_count(pidx)
    t1 = _rel_transform(xp, w1e)
    acc1 = _sc_aggregate(t1.reshape(4 * _NP, _D), pidx)
    h, cnt2d = _normalize(acc1, cntp, t1, b1r)
    t2 = _rel_transform(h, w2e)
    acc2 = _sc_aggregate(t2.reshape(4 * _NP, _D), pidx)
    return _pool(acc2, cnt2d, t2, b2r, batch3d)


# speedup vs baseline: 10.6419x; 1.0075x over previous
import functools

import jax
import jax.numpy as jnp
from jax import lax
from jax.experimental import pallas as pl
from jax.experimental.pallas import tpu as pltpu
from jax.experimental.pallas import tpu_sc as plsc

_N = 10000
_E = 320000
_D = 128
_G = 64
_NP = 10240
_NC = 2
_NS = 16
_NW = _NC * _NS
_EW = _E // _NW
_C = 32
_NCH = 320
_EP = _NW * _NCH * _C
_RPT = _NP // _NS
_BLK = 512
_NBLK = _NP // _BLK


def _mm_body(x_ref, w_ref, o_ref):
    o_ref[0] = jnp.dot(x_ref[...], w_ref[0], preferred_element_type=jnp.float32)


def _rel_transform(x, w_ext):
    return pl.pallas_call(
        _mm_body,
        grid=(_NBLK, 4),
        in_specs=[
            pl.BlockSpec((_BLK, _D), lambda i, r: (i, 0)),
            pl.BlockSpec((1, _D, _D), lambda i, r: (r, 0, 0)),
        ],
        out_specs=pl.BlockSpec((1, _BLK, _D), lambda i, r: (r, i, 0)),
        out_shape=jax.ShapeDtypeStruct((4, _NP, _D), jnp.float32),
    )(x, w_ext)


_NB = 4
_HCH = _NCH // 2


def _sc_body(table, pidx_hbm, acc_out,
             pkbuf, gb2, db2, rowbuf, acc_sh, gsem, ssem):
    c = lax.axis_index("c")
    s = lax.axis_index("s")
    wid = s * _NC + c

    zero16 = jnp.zeros((16,), jnp.float32)

    def _zrow(r, carry):
        def _zcol(k, carry2):
            rowbuf[0, r, pl.ds(k * 16, 16)] = zero16
            return carry2
        return lax.fori_loop(0, _D // 16, _zcol, carry)
    lax.fori_loop(0, _C, _zrow, 0)

    rows0 = s * _RPT
    for q in range(_RPT // _C):
        pltpu.sync_copy(rowbuf.at[0], acc_sh.at[pl.ds(rows0 + q * _C, _C)])
    plsc.subcore_barrier()

    def _unpack_to(r, b):
        for k in range(_C // 16):
            v = pkbuf[r, pl.ds(k * 16, 16)]
            gb2[b, pl.ds(k * 16, 16)] = lax.shift_right_logical(v, 14)
            db2[b, pl.ds(k * 16, 16)] = lax.bitwise_and(v, 16383)

    def _gather(b):
        pltpu.async_copy(table.at[gb2.at[b]], rowbuf.at[b], gsem.at[b])

    def _gwait(b):
        pltpu.make_async_copy(table.at[gb2.at[b]], rowbuf.at[b],
                              gsem.at[b]).wait()

    def _scatter(b):
        pltpu.async_copy(rowbuf.at[b], acc_sh.at[db2.at[b]], ssem.at[b],
                         add=True)

    def _swait(b):
        pltpu.make_async_copy(rowbuf.at[b], acc_sh.at[db2.at[b]],
                              ssem.at[b]).wait()

    for half in range(2):
        pltpu.sync_copy(pidx_hbm.at[wid, pl.ds(half * _HCH, _HCH)], pkbuf)

        def _step(i, carry):
            b = lax.rem(i, _NB)
            bc = lax.rem(i + 2, _NB)

            @pl.when(i < _HCH)
            def _():
                @pl.when(i >= _NB)
                def _():
                    _swait(b)
                _unpack_to(i, b)
                _gather(b)

            @pl.when(i >= 2)
            def _():
                _gwait(bc)
                _scatter(bc)
            return carry
        lax.fori_loop(0, _HCH + 2, _step, 0)

        def _drain(b, carry):
            _swait(b)
            return carry
        lax.fori_loop(0, _NB, _drain, 0)

    plsc.subcore_barrier()

    for q in range(_RPT // _C):
        pltpu.sync_copy(acc_sh.at[pl.ds(rows0 + q * _C, _C)], rowbuf.at[0])
        pltpu.sync_copy(rowbuf.at[0],
                        acc_out.at[pl.ds(c * _NP + rows0 + q * _C, _C)])


def _sc_aggregate(table_flat, pidx):
    mesh = plsc.VectorSubcoreMesh(core_axis_name="c", subcore_axis_name="s")
    k = functools.partial(
        pl.kernel,
        mesh=mesh,
        out_type=jax.ShapeDtypeStruct((_NC * _NP, _D), jnp.float32),
        scratch_types=[
            pltpu.VMEM((_HCH, _C), jnp.int32),
            pltpu.VMEM((_NB, _C), jnp.int32),
            pltpu.VMEM((_NB, _C), jnp.int32),
            pltpu.VMEM((_NB, _C, _D), jnp.float32),
            pltpu.VMEM_SHARED((_NP, _D), jnp.float32),
            pltpu.SemaphoreType.DMA((_NB,)),
            pltpu.SemaphoreType.DMA((_NB,)),
        ],
    )(_sc_body)
    return k(table_flat, pidx)


_CC = 64
_CNH = _EP // _NW // _CC


def _cnt_body(pidx_hbm, cnt_out, dbuf, onesbuf, cbuf, ccomp, cnt_sh):
    c = lax.axis_index("c")
    s = lax.axis_index("s")
    wid = s * _NC + c

    zero16 = jnp.zeros((16,), jnp.float32)
    ones16 = jnp.ones((16,), jnp.float32)

    def _zc(r, carry):
        def _zk(k, carry2):
            cbuf[r, pl.ds(k * 16, 16)] = zero16
            onesbuf[r, pl.ds(k * 16, 16)] = ones16
            return carry2
        return lax.fori_loop(0, _D // 16, _zk, carry)
    lax.fori_loop(0, _CC, _zc, 0)

    rows0 = s * _RPT
    for q in range(_RPT // _CC):
        pltpu.sync_copy(cbuf, cnt_sh.at[pl.ds(rows0 + q * _CC, _CC)])
    plsc.subcore_barrier()

    def _chunk(j, carry):
        pltpu.sync_copy(onesbuf, cnt_sh.at[dbuf.at[j]], add=True)
        return carry

    def _unpack(t, carry):
        r = t // (_CC // 16)
        k = t % (_CC // 16)
        v = dbuf[r, pl.ds(k * 16, 16)]
        dbuf[r, pl.ds(k * 16, 16)] = lax.bitwise_and(v, 16383)
        return carry

    pltpu.sync_copy(pidx_hbm.at[wid], dbuf)
    lax.fori_loop(0, _CNH * (_CC // 16), _unpack, 0)
    lax.fori_loop(0, _CNH, _chunk, 0)

    plsc.subcore_barrier()

    for q in range(_RPT // _CC):
        pltpu.sync_copy(cnt_sh.at[pl.ds(rows0 + q * _CC, _CC)], cbuf)

        def _ext(g, carry):
            lane = lax.iota(jnp.int32, 16)
            w = jnp.zeros((16,), jnp.float32)
            for i in range(16):
                v = cbuf[g * 16 + i, pl.ds(0, 16)]
                w = jnp.where(lane == i, v, w)
            ccomp[pl.ds(q * _CC + g * 16, 16)] = w
            return carry
        lax.fori_loop(0, _CC // 16, _ext, 0)
    pltpu.sync_copy(ccomp, cnt_out.at[pl.ds(c * _NP + rows0, _RPT)])


def _sc_count(pidx64):
    mesh = plsc.VectorSubcoreMesh(core_axis_name="c", subcore_axis_name="s")
    k = functools.partial(
        pl.kernel,
        mesh=mesh,
        out_type=jax.ShapeDtypeStruct((_NC * _NP,), jnp.float32),
        scratch_types=[
            pltpu.VMEM((_CNH, _CC), jnp.int32),
            pltpu.VMEM((_CC, _D), jnp.float32),
            pltpu.VMEM((_CC, _D), jnp.float32),
            pltpu.VMEM((_RPT,), jnp.float32),
            pltpu.VMEM_SHARED((_NP, _D), jnp.float32),
        ],
    )(_cnt_body)
    return k(pidx64)


def _norm_body(acc_ref, cntp_ref, xr_ref, b_ref, h_ref, cnt_ref):
    i = pl.program_id(0)
    row = lax.broadcasted_iota(jnp.int32, (_BLK, 1), 0) + i * _BLK
    cnt = cntp_ref[0] + cntp_ref[1]
    mean = (acc_ref[0] + acc_ref[1]) / jnp.maximum(cnt, 1.0)[:, None]
    h = jnp.maximum(mean + xr_ref[0] + b_ref[...], 0.0)
    h_ref[...] = jnp.where(row < _N, h, 0.0)
    cnt_ref[...] = cnt[None, :]


def _normalize(acc, cntp, t, b):
    return pl.pallas_call(
        _norm_body,
        grid=(_NBLK,),
        in_specs=[
            pl.BlockSpec((2, _BLK, _D), lambda i: (0, i, 0)),
            pl.BlockSpec((2, _BLK), lambda i: (0, i)),
            pl.BlockSpec((1, _BLK, _D), lambda i: (3, i, 0)),
            pl.BlockSpec((1, _D), lambda i: (0, 0)),
        ],
        out_specs=[
            pl.BlockSpec((_BLK, _D), lambda i: (i, 0)),
            pl.BlockSpec((1, _BLK), lambda i: (0, i)),
        ],
        out_shape=[
            jax.ShapeDtypeStruct((_NP, _D), jnp.float32),
            jax.ShapeDtypeStruct((1, _NP), jnp.float32),
        ],
    )(acc.reshape(2, _NP, _D), cntp.reshape(2, _NP), t, b)


def _pool_body(acc_ref, cnt_ref, hr_ref, b_ref, batch_ref, out_ref, counts_ref):
    i = pl.program_id(0)
    row = lax.broadcasted_iota(jnp.int32, (_BLK, 1), 0) + i * _BLK
    cnt = cnt_ref[0]
    h2 = jnp.maximum(
        (acc_ref[0] + acc_ref[1]) / jnp.maximum(cnt, 1.0)[:, None]
        + hr_ref[0] + b_ref[...], 0.0)
    h2 = jnp.where(row < _N, h2, 0.0)
    gid = lax.broadcasted_iota(jnp.int32, (_G, _BLK), 0)
    onehot = (batch_ref[0, 0][None, :] == gid).astype(jnp.float32)

    @pl.when(i == 0)
    def _():
        out_ref[...] = jnp.zeros_like(out_ref)
        counts_ref[...] = jnp.zeros_like(counts_ref)

    out_ref[...] += jnp.dot(onehot, h2, preferred_element_type=jnp.float32)
    counts_ref[...] += jnp.sum(onehot, axis=1)[:, None]

    @pl.when(i == pl.num_programs(0) - 1)
    def _():
        out_ref[...] = out_ref[...] / jnp.maximum(counts_ref[...], 1.0)


def _pool(acc, cnt2d, t, b, batch3d):
    return pl.pallas_call(
        _pool_body,
        grid=(_NBLK,),
        in_specs=[
            pl.BlockSpec((2, _BLK, _D), lambda i: (0, i, 0)),
            pl.BlockSpec((1, _BLK), lambda i: (0, i)),
            pl.BlockSpec((1, _BLK, _D), lambda i: (3, i, 0)),
            pl.BlockSpec((1, _D), lambda i: (0, 0)),
            pl.BlockSpec((1, 1, _BLK), lambda i: (i, 0, 0)),
        ],
        out_specs=pl.BlockSpec((_G, _D), lambda i: (0, 0)),
        out_shape=jax.ShapeDtypeStruct((_G, _D), jnp.float32),
        scratch_shapes=[pltpu.VMEM((_G, _D), jnp.float32)],
    )(acc.reshape(2, _NP, _D), cnt2d, t, b, batch3d)


def kernel(x, edge_index, edge_type, batch, W1, root1, b1, W2, root2, b2):
    xp = jnp.pad(x, ((0, _NP - _N), (0, 0)))
    src = edge_index[0].astype(jnp.int32)
    dst = edge_index[1].astype(jnp.int32)
    et = edge_type.astype(jnp.int32)
    packed = (et * _NP + src) * 16384 + dst
    pad_dst = _N + jnp.arange(_EP - _E, dtype=jnp.int32) % (_NP - _N)
    pidx = jnp.concatenate([packed, pad_dst]).reshape(_NW, _NCH, _C)
    batch3d = jnp.concatenate(
        [batch.astype(jnp.int32), jnp.full((_NP - _N,), _G, jnp.int32)]
    ).reshape(_NBLK, 1, _BLK)
    w1e = jnp.concatenate([W1, root1[None]], axis=0)
    w2e = jnp.concatenate([W2, root2[None]], axis=0)
    b1r = b1.reshape(1, _D)
    b2r = b2.reshape(1, _D)

    cntp = _sc_count(pidx.reshape(_NW, _CNH, _CC))
    t1 = _rel_transform(xp, w1e)
    acc1 = _sc_aggregate(t1.reshape(4 * _NP, _D), pidx)
    h, cnt2d = _normalize(acc1, cntp, t1, b1r)
    t2 = _rel_transform(h, w2e)
    acc2 = _sc_aggregate(t2.reshape(4 * _NP, _D), pidx)
    return _pool(acc2, cnt2d, t2, b2r, batch3d)
